# baseline (speedup 1.0000x reference)
import functools

import jax
import jax.numpy as jnp
from jax import lax
from jax.experimental import pallas as pl
from jax.experimental.pallas import tpu as pltpu
from jax.experimental.pallas import tpu_sc as plsc

D, H, W = 11, 100, 88
C_IN = 64
NVOX = D * H * W
N_ROWS = 40000
N_PADROWS = 32
PH, PW = H + 2, W + 2
PS = PH * PW
PD = D + 2
PVOX = PD * PS
NW = 32
WSZ = 3736
PV_PAD = NW * WSZ
CHUNK = 1024
NCHUNK = 4
GATHER_SUB = 128


def _sc_scatter(indices, features_pad):
    mesh = plsc.VectorSubcoreMesh(core_axis_name="c", subcore_axis_name="s")

    @functools.partial(
        pl.kernel, mesh=mesh,
        compiler_params=pltpu.CompilerParams(
            use_tc_tiling_on_sc=False, needs_layout_passes=False),
        out_type=[
            jax.ShapeDtypeStruct((PV_PAD, C_IN), jnp.float32),
            jax.ShapeDtypeStruct((PV_PAD,), jnp.float32),
        ],
        scratch_types=[
            pltpu.VMEM((N_ROWS,), jnp.int32),
            pltpu.VMEM((WSZ,), jnp.int32),
            pltpu.VMEM((CHUNK,), jnp.int32),
            pltpu.VMEM((CHUNK, C_IN), jnp.float32),
            pltpu.VMEM((CHUNK,), jnp.float32),
            pltpu.VMEM((16,), jnp.int32),
            pltpu.SemaphoreType.DMA,
        ],
    )
    def k(starts_hbm, idx_hbm, feat_hbm, dense_hbm, mask_hbm, idx_v, win_v,
          gidx_v, rows_v, mval_v, startv_v, sem):
        wid = lax.axis_index("s") * 2 + lax.axis_index("c")
        start = wid * WSZ
        lane = lax.iota(jnp.int32, 16)

        pltpu.sync_copy(starts_hbm.at[pl.ds(wid * 16, 16)], startv_v)
        pltpu.sync_copy(idx_hbm, idx_v)
        startv = startv_v[...]

        def init_body(i, _):
            win_v[pl.ds(i * 16, 16)] = jnp.full((16,), -1, jnp.int32)
            return 0
        lax.fori_loop(0, WSZ // 16, init_body, 0)
        win_v[pl.ds(WSZ - 16, 16)] = jnp.full((16,), -1, jnp.int32)

        def g_body(g, rv):
            b = g * 16
            vi = idx_v[pl.ds(b, 16)]
            d = jnp.zeros((16,), jnp.int32)
            one = jnp.ones((16,), jnp.int32)
            zero = jnp.zeros((16,), jnp.int32)
            for kk in range(1, D):
                d = d + jnp.where(vi >= kk * (H * W), one, zero)
            rem = vi - d * (H * W)
            h = (rem * 47663) >> 22
            w = rem - h * W
            pos = d * PS + h * PW + w + (PS + PW + 1)
            local = pos - startv
            inr = (local >= 0) & (local < WSZ)
            slocc = jnp.where(inr, local, zero)
            cur0 = plsc.load_gather(win_v, [slocc])
            pend0 = jnp.where(inr & (rv > cur0), one, zero)

            def w_cond(pend):
                return jnp.max(pend) > 0

            def w_body(pend):
                plsc.store_scatter(win_v, [slocc], rv, mask=pend > 0)
                cur = plsc.load_gather(win_v, [slocc])
                return jnp.where(inr & (rv > cur), one, zero)

            lax.while_loop(w_cond, w_body, pend0)
            return rv + 16
        lax.fori_loop(0, N_ROWS // 16, g_body, lane)

        def c_body(c, _):
            lb = jnp.minimum(c * CHUNK, WSZ - CHUNK)
            cb = start + lb

            def b_body(j, _):
                o = j * 16
                wv = win_v[pl.ds(lb + o, 16)]
                ok = wv >= 0
                padrow = N_ROWS + ((o + lane) & (N_PADROWS - 1))
                gidx_v[pl.ds(o, 16)] = jnp.where(ok, wv, padrow)
                mval_v[pl.ds(o, 16)] = jnp.where(
                    ok, jnp.float32(1.0), jnp.float32(0.0))
                return 0
            lax.fori_loop(0, CHUNK // 16, b_body, 0)

            cps = [
                pltpu.async_copy(
                    feat_hbm.at[gidx_v.at[pl.ds(s * GATHER_SUB, GATHER_SUB)]],
                    rows_v.at[pl.ds(s * GATHER_SUB, GATHER_SUB)], sem)
                for s in range(CHUNK // GATHER_SUB)
            ]
            for cp in cps:
                cp.wait()
            pltpu.sync_copy(rows_v, dense_hbm.at[pl.ds(cb, CHUNK)])
            pltpu.sync_copy(mval_v, mask_hbm.at[pl.ds(cb, CHUNK)])
            return 0
        lax.fori_loop(0, NCHUNK, c_body, 0)

    starts = jnp.repeat(jnp.arange(NW, dtype=jnp.int32) * WSZ, 16)
    return k(starts, indices, features_pad)


def _shift_acc(spad_ref, taps=9):
    acc = None
    for t in range(taps):
        kh, kw = t // 3, t % 3
        delta = (kh - 1) * PW + (kw - 1)
        sl = spad_ref[pl.ds(t * 32, 32), pl.ds(96 + delta, PS)]
        acc = sl if acc is None else acc + sl
    return acc


def _conv3_kernel(x0, x1, x2, w0, w1, w2, spad_ref):
    b = jnp.bfloat16
    s = (jnp.dot(w0, x0.astype(b), preferred_element_type=jnp.float32)
         + jnp.dot(w1, x1.astype(b), preferred_element_type=jnp.float32)
         + jnp.dot(w2, x2.astype(b), preferred_element_type=jnp.float32))
    spad_ref[:, pl.ds(96, PS)] = s
    return _shift_acc(spad_ref)


def _zero_halo(spad_ref):
    spad_ref[:, pl.ds(0, 96)] = jnp.zeros((288, 96), jnp.float32)
    spad_ref[:, pl.ds(96 + PS, 96)] = jnp.zeros((288, 96), jnp.float32)


def _stage_a(xpad, mpad, w1t, gscale, beta):
    def body(x0_ref, x1_ref, x2_ref, m_ref, w_ref, g_ref, b_ref, o_ref,
             spad_ref):
        @pl.when(pl.program_id(0) == 0)
        def _():
            _zero_halo(spad_ref)

        acc = _conv3_kernel(x0_ref[0], x1_ref[0], x2_ref[0],
                            w_ref[0], w_ref[1], w_ref[2], spad_ref)
        y = jnp.maximum(acc * g_ref[...] + b_ref[...], 0.0) * m_ref[0]
        o_ref[0] = y

    xspec = lambda j: pl.BlockSpec((1, C_IN, PS), lambda d, j=j: (d + j, 0, 0))
    return pl.pallas_call(
        body,
        grid=(D,),
        in_specs=[
            xspec(0), xspec(1), xspec(2),
            pl.BlockSpec((1, 1, PS), lambda d: (d + 1, 0, 0)),
            pl.BlockSpec((3, 288, C_IN), lambda d: (0, 0, 0)),
            pl.BlockSpec((32, 1), lambda d: (0, 0)),
            pl.BlockSpec((32, 1), lambda d: (0, 0)),
        ],
        out_specs=pl.BlockSpec((1, 32, PS), lambda d: (d, 0, 0)),
        out_shape=jax.ShapeDtypeStruct((D, 32, PS), jnp.float32),
        scratch_shapes=[pltpu.VMEM((288, PS + 192), jnp.float32)],
    )(xpad, xpad, xpad, mpad, w1t, gscale, beta)


def _stage_b(y1p, mpad, w4t):
    def body(x0_ref, x1_ref, x2_ref, m0_ref, m1_ref, m2_ref, w_ref,
             o_ref, mo_ref):
        s = pl.program_id(0)
        interior = jnp.logical_and(s > 0, s < 6)

        @pl.when(interior)
        def _():
            acc = (jnp.dot(w_ref[0], x0_ref[0],
                           preferred_element_type=jnp.float32)
                   + jnp.dot(w_ref[1], x1_ref[0],
                             preferred_element_type=jnp.float32)
                   + jnp.dot(w_ref[2], x2_ref[0],
                             preferred_element_type=jnp.float32))
            o_ref[0] = acc
            mo_ref[0] = jnp.maximum(jnp.maximum(m0_ref[0], m1_ref[0]),
                                    m2_ref[0])

        @pl.when(jnp.logical_not(interior))
        def _():
            o_ref[0] = jnp.zeros((32, PS), jnp.float32)
            mo_ref[0] = jnp.zeros((1, PS), jnp.float32)

    yspec = lambda k: pl.BlockSpec(
        (1, 32, PS),
        lambda s, k=k: (jnp.clip(2 * s - 2 + k, 0, D - 1), 0, 0))
    mspec = lambda k: pl.BlockSpec(
        (1, 1, PS),
        lambda s, k=k: (jnp.clip(2 * s - 1 + k, 0, PD - 1), 0, 0))
    return pl.pallas_call(
        body,
        grid=(7,),
        in_specs=[
            yspec(0), yspec(1), yspec(2),
            mspec(0), mspec(1), mspec(2),
            pl.BlockSpec((3, 32, 32), lambda s: (0, 0, 0)),
        ],
        out_specs=[
            pl.BlockSpec((1, 32, PS), lambda s: (s, 0, 0)),
            pl.BlockSpec((1, 1, PS), lambda s: (s, 0, 0)),
        ],
        out_shape=[
            jax.ShapeDtypeStruct((7, 32, PS), jnp.float32),
            jax.ShapeDtypeStruct((7, 1, PS), jnp.float32),
        ],
    )(y1p, y1p, y1p, mpad, mpad, mpad, w4t)


def _stage_c(ypadded, m2p, wt, nout, dpad_out):
    def body(x0_ref, x1_ref, x2_ref, m_ref, w_ref, o_ref, spad_ref):
        s = pl.program_id(0)

        @pl.when(s == 0)
        def _():
            _zero_halo(spad_ref)

        if dpad_out:
            interior = jnp.logical_and(s > 0, s < nout - 1)

            @pl.when(interior)
            def _():
                acc = _conv3_kernel(x0_ref[0], x1_ref[0], x2_ref[0],
                                    w_ref[0], w_ref[1], w_ref[2], spad_ref)
                o_ref[0] = acc * m_ref[0]

            @pl.when(jnp.logical_not(interior))
            def _():
                o_ref[0] = jnp.zeros((32, PS), jnp.float32)
        else:
            acc = _conv3_kernel(x0_ref[0], x1_ref[0], x2_ref[0],
                                w_ref[0], w_ref[1], w_ref[2], spad_ref)
            o_ref[0] = acc * m_ref[0]

    if dpad_out:
        xidx = lambda k: (lambda s, k=k: (jnp.clip(s - 1 + k, 0, 6), 0, 0))
        midx = lambda s: (jnp.clip(s, 0, 6), 0, 0)
    else:
        xidx = lambda k: (lambda s, k=k: (s + k, 0, 0))
        midx = lambda s: (s + 1, 0, 0)

    return pl.pallas_call(
        body,
        grid=(nout,),
        in_specs=[
            pl.BlockSpec((1, 32, PS), xidx(0)),
            pl.BlockSpec((1, 32, PS), xidx(1)),
            pl.BlockSpec((1, 32, PS), xidx(2)),
            pl.BlockSpec((1, 1, PS), midx),
            pl.BlockSpec((3, 288, 32), lambda s: (0, 0, 0)),
        ],
        out_specs=pl.BlockSpec((1, 32, PS), lambda s: (s, 0, 0)),
        out_shape=jax.ShapeDtypeStruct((nout, 32, PS), jnp.float32),
        scratch_shapes=[pltpu.VMEM((288, PS + 192), jnp.float32)],
    )(ypadded, ypadded, ypadded, m2p, wt)


def _stage_e(y6, w7t):
    def body(x0_ref, x1_ref, x2_ref, w_ref, o_ref):
        acc = (jnp.dot(w_ref[0], x0_ref[0], preferred_element_type=jnp.float32)
               + jnp.dot(w_ref[1], x1_ref[0],
                         preferred_element_type=jnp.float32)
               + jnp.dot(w_ref[2], x2_ref[0],
                         preferred_element_type=jnp.float32))
        o_ref[0] = acc

    yspec = lambda k: pl.BlockSpec((1, 32, PS),
                                   lambda t, k=k: (2 * t + k, 0, 0))
    return pl.pallas_call(
        body,
        grid=(2,),
        in_specs=[yspec(0), yspec(1), yspec(2),
                  pl.BlockSpec((3, 32, 32), lambda t: (0, 0, 0))],
        out_specs=pl.BlockSpec((1, 32, PS), lambda t: (t, 0, 0)),
        out_shape=jax.ShapeDtypeStruct((2, 32, PS), jnp.float32),
    )(y6, y6, y6, w7t)


def kernel(features, indices, w1, gamma, beta, w4, w5, w6, w7):
    features_pad = jnp.concatenate(
        [features, jnp.zeros((N_PADROWS, C_IN), jnp.float32)], axis=0)
    dense_flat, mask_flat = _sc_scatter(indices, features_pad)
    xpad = jnp.transpose(dense_flat[:PVOX].reshape(PD, PS, C_IN), (0, 2, 1))
    mpad = mask_flat[:PVOX].reshape(PD, 1, PS)

    w1t = jnp.transpose(w1, (0, 1, 2, 4, 3)).reshape(3, 288, C_IN)
    w1t = w1t.astype(jnp.bfloat16)
    w5t = jnp.transpose(w5, (0, 1, 2, 4, 3)).reshape(3, 288, 32)
    w5t = w5t.astype(jnp.bfloat16)
    w6t = jnp.transpose(w6, (0, 1, 2, 4, 3)).reshape(3, 288, 32)
    w6t = w6t.astype(jnp.bfloat16)
    w4t = jnp.transpose(w4.reshape(3, 32, 32), (0, 2, 1))
    w7t = jnp.transpose(w7.reshape(3, 32, 32), (0, 2, 1))
    gscale = (gamma / jnp.sqrt(jnp.float32(1.0 + 1e-3))).reshape(32, 1)
    beta2 = beta.reshape(32, 1)

    y1p = _stage_a(xpad, mpad, w1t, gscale, beta2)
    y4p, m2p = _stage_b(y1p, mpad, w4t)
    y5p = _stage_c(y4p, m2p, w5t, 7, True)
    y6 = _stage_c(y5p, m2p, w6t, 5, False)
    y7 = _stage_e(y6, w7t)

    out = y7.reshape(2, 32, PH, PW)[:, :, 1:H + 1, 1:W + 1]
    return jnp.transpose(out, (1, 0, 2, 3))[None]


# speedup vs baseline: 1.1691x; 1.1691x over previous
import functools

import jax
import jax.numpy as jnp
from jax import lax
from jax.experimental import pallas as pl
from jax.experimental.pallas import tpu as pltpu
from jax.experimental.pallas import tpu_sc as plsc

D, H, W = 11, 100, 88
C_IN = 64
NVOX = D * H * W
N_ROWS = 40000
N_PADROWS = 256
PH, PW = H + 2, W + 2
PS = PH * PW
PD = D + 2
PVOX = PD * PS
NW = 32
WSZ = 3736
PV_PAD = NW * WSZ
CHUNK = 1024
NCHUNK = 4
GATHER_SUB = 128


def _sc_scatter(indices, features_pad):
    mesh = plsc.VectorSubcoreMesh(core_axis_name="c", subcore_axis_name="s")

    @functools.partial(
        pl.kernel, mesh=mesh,
        compiler_params=pltpu.CompilerParams(
            use_tc_tiling_on_sc=False, needs_layout_passes=False),
        out_type=[
            jax.ShapeDtypeStruct((PV_PAD, C_IN), jnp.float32),
            jax.ShapeDtypeStruct((PV_PAD,), jnp.float32),
        ],
        scratch_types=[
            pltpu.VMEM((N_ROWS,), jnp.int32),
            pltpu.VMEM((WSZ,), jnp.int32),
            pltpu.VMEM((CHUNK,), jnp.int32),
            pltpu.VMEM((CHUNK, C_IN), jnp.float32),
            pltpu.VMEM((CHUNK,), jnp.float32),
            pltpu.VMEM((16,), jnp.int32),
            pltpu.SemaphoreType.DMA,
        ],
    )
    def k(starts_hbm, idx_hbm, feat_hbm, dense_hbm, mask_hbm, idx_v, win_v,
          gidx_v, rows_v, mval_v, startv_v, sem):
        wid = lax.axis_index("s") * 2 + lax.axis_index("c")
        start = wid * WSZ
        lane = lax.iota(jnp.int32, 16)

        pltpu.sync_copy(starts_hbm.at[pl.ds(wid * 16, 16)], startv_v)
        pltpu.sync_copy(idx_hbm, idx_v)
        startv = startv_v[...]

        def init_body(i, _):
            win_v[pl.ds(i * 16, 16)] = jnp.full((16,), -1, jnp.int32)
            return 0
        lax.fori_loop(0, WSZ // 16, init_body, 0)
        win_v[pl.ds(WSZ - 16, 16)] = jnp.full((16,), -1, jnp.int32)

        def g_body(g, rv):
            b = g * 16
            vi = idx_v[pl.ds(b, 16)]
            d = jnp.zeros((16,), jnp.int32)
            one = jnp.ones((16,), jnp.int32)
            zero = jnp.zeros((16,), jnp.int32)
            for kk in range(1, D):
                d = d + jnp.where(vi >= kk * (H * W), one, zero)
            rem = vi - d * (H * W)
            h = (rem * 47663) >> 22
            w = rem - h * W
            pos = d * PS + h * PW + w + (PS + PW + 1)
            local = pos - startv
            inr = (local >= 0) & (local < WSZ)
            slocc = jnp.where(inr, local, zero)
            plsc.store_scatter(win_v, [slocc], rv, mask=inr)
            cur0 = plsc.load_gather(win_v, [slocc])
            pend0 = jnp.where(inr & (rv > cur0), one, zero)

            def w_cond(pend):
                return jnp.max(pend) > 0

            def w_body(pend):
                plsc.store_scatter(win_v, [slocc], rv, mask=pend > 0)
                cur = plsc.load_gather(win_v, [slocc])
                return jnp.where(inr & (rv > cur), one, zero)

            lax.while_loop(w_cond, w_body, pend0)
            return rv + 16
        lax.fori_loop(0, N_ROWS // 16, g_body, lane)

        def c_body(c, _):
            lb = jnp.minimum(c * CHUNK, WSZ - CHUNK)
            cb = start + lb

            def b_body(j, _):
                o = j * 16
                wv = win_v[pl.ds(lb + o, 16)]
                ok = wv >= 0
                padrow = N_ROWS + ((o + lane) & (N_PADROWS - 1))
                gidx_v[pl.ds(o, 16)] = jnp.where(ok, wv, padrow)
                mval_v[pl.ds(o, 16)] = jnp.where(
                    ok, jnp.float32(1.0), jnp.float32(0.0))
                return 0
            lax.fori_loop(0, CHUNK // 16, b_body, 0)

            cps = [
                pltpu.async_copy(
                    feat_hbm.at[gidx_v.at[pl.ds(s * GATHER_SUB, GATHER_SUB)]],
                    rows_v.at[pl.ds(s * GATHER_SUB, GATHER_SUB)], sem)
                for s in range(CHUNK // GATHER_SUB)
            ]
            for cp in cps:
                cp.wait()
            pltpu.sync_copy(rows_v, dense_hbm.at[pl.ds(cb, CHUNK)])
            pltpu.sync_copy(mval_v, mask_hbm.at[pl.ds(cb, CHUNK)])
            return 0
        lax.fori_loop(0, NCHUNK, c_body, 0)

    starts = jnp.repeat(jnp.arange(NW, dtype=jnp.int32) * WSZ, 16)
    return k(starts, indices, features_pad)


def _shift_acc(spad_ref, taps=9):
    acc = None
    for t in range(taps):
        kh, kw = t // 3, t % 3
        delta = (kh - 1) * PW + (kw - 1)
        sl = spad_ref[pl.ds(t * 32, 32), pl.ds(96 + delta, PS)]
        acc = sl if acc is None else acc + sl
    return acc


def _conv3_kernel(x0, x1, x2, w0, w1, w2, spad_ref):
    s = (jnp.dot(w0, x0, preferred_element_type=jnp.float32)
         + jnp.dot(w1, x1, preferred_element_type=jnp.float32)
         + jnp.dot(w2, x2, preferred_element_type=jnp.float32))
    spad_ref[:, pl.ds(96, PS)] = s
    return _shift_acc(spad_ref)


def _zero_halo(spad_ref):
    spad_ref[:, pl.ds(0, 96)] = jnp.zeros((288, 96), jnp.float32)
    spad_ref[:, pl.ds(96 + PS, 96)] = jnp.zeros((288, 96), jnp.float32)


def _stage_a(xpad, mpad, w1t, gscale, beta):
    def body(x0_ref, x1_ref, x2_ref, m_ref, w_ref, g_ref, b_ref, o_ref,
             spad_ref):
        @pl.when(pl.program_id(0) == 0)
        def _():
            _zero_halo(spad_ref)

        acc = _conv3_kernel(x0_ref[0], x1_ref[0], x2_ref[0],
                            w_ref[0], w_ref[1], w_ref[2], spad_ref)
        y = jnp.maximum(acc * g_ref[...] + b_ref[...], 0.0) * m_ref[0]
        o_ref[0] = y

    xspec = lambda j: pl.BlockSpec((1, C_IN, PS), lambda d, j=j: (d + j, 0, 0))
    return pl.pallas_call(
        body,
        grid=(D,),
        in_specs=[
            xspec(0), xspec(1), xspec(2),
            pl.BlockSpec((1, 1, PS), lambda d: (d + 1, 0, 0)),
            pl.BlockSpec((3, 288, C_IN), lambda d: (0, 0, 0)),
            pl.BlockSpec((32, 1), lambda d: (0, 0)),
            pl.BlockSpec((32, 1), lambda d: (0, 0)),
        ],
        out_specs=pl.BlockSpec((1, 32, PS), lambda d: (d, 0, 0)),
        out_shape=jax.ShapeDtypeStruct((D, 32, PS), jnp.float32),
        scratch_shapes=[pltpu.VMEM((288, PS + 192), jnp.float32)],
    )(xpad, xpad, xpad, mpad, w1t, gscale, beta)


def _stage_b(y1p, mpad, w4t):
    def body(x0_ref, x1_ref, x2_ref, m0_ref, m1_ref, m2_ref, w_ref,
             o_ref, mo_ref):
        s = pl.program_id(0)
        interior = jnp.logical_and(s > 0, s < 6)

        @pl.when(interior)
        def _():
            acc = (jnp.dot(w_ref[0], x0_ref[0],
                           preferred_element_type=jnp.float32)
                   + jnp.dot(w_ref[1], x1_ref[0],
                             preferred_element_type=jnp.float32)
                   + jnp.dot(w_ref[2], x2_ref[0],
                             preferred_element_type=jnp.float32))
            o_ref[0] = acc
            mo_ref[0] = jnp.maximum(jnp.maximum(m0_ref[0], m1_ref[0]),
                                    m2_ref[0])

        @pl.when(jnp.logical_not(interior))
        def _():
            o_ref[0] = jnp.zeros((32, PS), jnp.float32)
            mo_ref[0] = jnp.zeros((1, PS), jnp.float32)

    yspec = lambda k: pl.BlockSpec(
        (1, 32, PS),
        lambda s, k=k: (jnp.clip(2 * s - 2 + k, 0, D - 1), 0, 0))
    mspec = lambda k: pl.BlockSpec(
        (1, 1, PS),
        lambda s, k=k: (jnp.clip(2 * s - 1 + k, 0, PD - 1), 0, 0))
    return pl.pallas_call(
        body,
        grid=(7,),
        in_specs=[
            yspec(0), yspec(1), yspec(2),
            mspec(0), mspec(1), mspec(2),
            pl.BlockSpec((3, 32, 32), lambda s: (0, 0, 0)),
        ],
        out_specs=[
            pl.BlockSpec((1, 32, PS), lambda s: (s, 0, 0)),
            pl.BlockSpec((1, 1, PS), lambda s: (s, 0, 0)),
        ],
        out_shape=[
            jax.ShapeDtypeStruct((7, 32, PS), jnp.float32),
            jax.ShapeDtypeStruct((7, 1, PS), jnp.float32),
        ],
    )(y1p, y1p, y1p, mpad, mpad, mpad, w4t)


def _stage_c(ypadded, m2p, wt, nout, dpad_out):
    def body(x0_ref, x1_ref, x2_ref, m_ref, w_ref, o_ref, spad_ref):
        s = pl.program_id(0)

        @pl.when(s == 0)
        def _():
            _zero_halo(spad_ref)

        if dpad_out:
            interior = jnp.logical_and(s > 0, s < nout - 1)

            @pl.when(interior)
            def _():
                acc = _conv3_kernel(x0_ref[0], x1_ref[0], x2_ref[0],
                                    w_ref[0], w_ref[1], w_ref[2], spad_ref)
                o_ref[0] = acc * m_ref[0]

            @pl.when(jnp.logical_not(interior))
            def _():
                o_ref[0] = jnp.zeros((32, PS), jnp.float32)
        else:
            acc = _conv3_kernel(x0_ref[0], x1_ref[0], x2_ref[0],
                                w_ref[0], w_ref[1], w_ref[2], spad_ref)
            o_ref[0] = acc * m_ref[0]

    if dpad_out:
        xidx = lambda k: (lambda s, k=k: (jnp.clip(s - 1 + k, 0, 6), 0, 0))
        midx = lambda s: (jnp.clip(s, 0, 6), 0, 0)
    else:
        xidx = lambda k: (lambda s, k=k: (s + k, 0, 0))
        midx = lambda s: (s + 1, 0, 0)

    return pl.pallas_call(
        body,
        grid=(nout,),
        in_specs=[
            pl.BlockSpec((1, 32, PS), xidx(0)),
            pl.BlockSpec((1, 32, PS), xidx(1)),
            pl.BlockSpec((1, 32, PS), xidx(2)),
            pl.BlockSpec((1, 1, PS), midx),
            pl.BlockSpec((3, 288, 32), lambda s: (0, 0, 0)),
        ],
        out_specs=pl.BlockSpec((1, 32, PS), lambda s: (s, 0, 0)),
        out_shape=jax.ShapeDtypeStruct((nout, 32, PS), jnp.float32),
        scratch_shapes=[pltpu.VMEM((288, PS + 192), jnp.float32)],
    )(ypadded, ypadded, ypadded, m2p, wt)


def _stage_e(y6, w7t):
    def body(x0_ref, x1_ref, x2_ref, w_ref, o_ref):
        acc = (jnp.dot(w_ref[0], x0_ref[0], preferred_element_type=jnp.float32)
               + jnp.dot(w_ref[1], x1_ref[0],
                         preferred_element_type=jnp.float32)
               + jnp.dot(w_ref[2], x2_ref[0],
                         preferred_element_type=jnp.float32))
        o_ref[0] = acc

    yspec = lambda k: pl.BlockSpec((1, 32, PS),
                                   lambda t, k=k: (2 * t + k, 0, 0))
    return pl.pallas_call(
        body,
        grid=(2,),
        in_specs=[yspec(0), yspec(1), yspec(2),
                  pl.BlockSpec((3, 32, 32), lambda t: (0, 0, 0))],
        out_specs=pl.BlockSpec((1, 32, PS), lambda t: (t, 0, 0)),
        out_shape=jax.ShapeDtypeStruct((2, 32, PS), jnp.float32),
    )(y6, y6, y6, w7t)


def kernel(features, indices, w1, gamma, beta, w4, w5, w6, w7):
    features_pad = jnp.concatenate(
        [features, jnp.zeros((N_PADROWS, C_IN), jnp.float32)], axis=0)
    dense_flat, mask_flat = _sc_scatter(indices, features_pad)
    xpad = jnp.transpose(dense_flat[:PVOX].reshape(PD, PS, C_IN), (0, 2, 1))
    mpad = mask_flat[:PVOX].reshape(PD, 1, PS)

    w1t = jnp.transpose(w1, (0, 1, 2, 4, 3)).reshape(3, 288, C_IN)
    w5t = jnp.transpose(w5, (0, 1, 2, 4, 3)).reshape(3, 288, 32)
    w6t = jnp.transpose(w6, (0, 1, 2, 4, 3)).reshape(3, 288, 32)
    w4t = jnp.transpose(w4.reshape(3, 32, 32), (0, 2, 1))
    w7t = jnp.transpose(w7.reshape(3, 32, 32), (0, 2, 1))
    gscale = (gamma / jnp.sqrt(jnp.float32(1.0 + 1e-3))).reshape(32, 1)
    beta2 = beta.reshape(32, 1)

    y1p = _stage_a(xpad, mpad, w1t, gscale, beta2)
    y4p, m2p = _stage_b(y1p, mpad, w4t)
    y5p = _stage_c(y4p, m2p, w5t, 7, True)
    y6 = _stage_c(y5p, m2p, w6t, 5, False)
    y7 = _stage_e(y6, w7t)

    out = y7.reshape(2, 32, PH, PW)[:, :, 1:H + 1, 1:W + 1]
    return jnp.transpose(out, (1, 0, 2, 3))[None]


# speedup vs baseline: 1.2167x; 1.0407x over previous
import functools

import jax
import jax.numpy as jnp
from jax import lax
from jax.experimental import pallas as pl
from jax.experimental.pallas import tpu as pltpu
from jax.experimental.pallas import tpu_sc as plsc

D, H, W = 11, 100, 88
C_IN = 64
NVOX = D * H * W
N_ROWS = 40000
N_PADROWS = 1024
PH, PW = H + 2, W + 2
PS = PH * PW
PD = D + 2
PVOX = PD * PS
NW = 32
WSZ = 3736
PV_PAD = NW * WSZ
CHUNK = 1024
NCHUNK = 4
GATHER_SUB = 128


def _sc_scatter(indices, features_pad):
    mesh = plsc.VectorSubcoreMesh(core_axis_name="c", subcore_axis_name="s")

    @functools.partial(
        pl.kernel, mesh=mesh,
        compiler_params=pltpu.CompilerParams(
            use_tc_tiling_on_sc=False, needs_layout_passes=False),
        out_type=[
            jax.ShapeDtypeStruct((PV_PAD, C_IN), jnp.float32),
            jax.ShapeDtypeStruct((PV_PAD,), jnp.float32),
        ],
        scratch_types=[
            pltpu.VMEM((N_ROWS,), jnp.int32),
            pltpu.VMEM((WSZ,), jnp.int32),
            pltpu.VMEM((CHUNK,), jnp.int32),
            pltpu.VMEM((CHUNK, C_IN), jnp.float32),
            pltpu.VMEM((CHUNK,), jnp.float32),
            pltpu.VMEM((16,), jnp.int32),
            pltpu.SemaphoreType.DMA,
        ],
    )
    def k(starts_hbm, idx_hbm, feat_hbm, dense_hbm, mask_hbm, idx_v, win_v,
          gidx_v, rows_v, mval_v, startv_v, sem):
        wid = lax.axis_index("s") * 2 + lax.axis_index("c")
        start = wid * WSZ
        lane = lax.iota(jnp.int32, 16)

        pltpu.sync_copy(starts_hbm.at[pl.ds(wid * 16, 16)], startv_v)
        pltpu.sync_copy(idx_hbm, idx_v)
        startv = startv_v[...]

        def init_body(i, _):
            win_v[pl.ds(i * 16, 16)] = jnp.full((16,), -1, jnp.int32)
            return 0
        lax.fori_loop(0, WSZ // 16, init_body, 0)
        win_v[pl.ds(WSZ - 16, 16)] = jnp.full((16,), -1, jnp.int32)

        def g_body(g, rv):
            b = g * 16
            vi = idx_v[pl.ds(b, 16)]
            d = jnp.zeros((16,), jnp.int32)
            one = jnp.ones((16,), jnp.int32)
            zero = jnp.zeros((16,), jnp.int32)
            for kk in range(1, D):
                d = d + jnp.where(vi >= kk * (H * W), one, zero)
            rem = vi - d * (H * W)
            h = (rem * 47663) >> 22
            w = rem - h * W
            pos = d * PS + h * PW + w + (PS + PW + 1)
            local = pos - startv
            inr = (local >= 0) & (local < WSZ)
            slocc = jnp.where(inr, local, zero)
            plsc.store_scatter(win_v, [slocc], rv, mask=inr)
            cur0 = plsc.load_gather(win_v, [slocc])
            pend0 = jnp.where(inr & (rv > cur0), one, zero)

            def w_cond(pend):
                return jnp.max(pend) > 0

            def w_body(pend):
                plsc.store_scatter(win_v, [slocc], rv, mask=pend > 0)
                cur = plsc.load_gather(win_v, [slocc])
                return jnp.where(inr & (rv > cur), one, zero)

            lax.while_loop(w_cond, w_body, pend0)
            return rv + 16
        lax.fori_loop(0, N_ROWS // 16, g_body, lane)

        def c_body(c, _):
            lb = jnp.minimum(c * CHUNK, WSZ - CHUNK)
            cb = start + lb

            def b_body(j, _):
                o = j * 16
                wv = win_v[pl.ds(lb + o, 16)]
                ok = wv >= 0
                padrow = N_ROWS + ((o + lane) & (N_PADROWS - 1))
                gidx_v[pl.ds(o, 16)] = jnp.where(ok, wv, padrow)
                mval_v[pl.ds(o, 16)] = jnp.where(
                    ok, jnp.float32(1.0), jnp.float32(0.0))
                return 0
            lax.fori_loop(0, CHUNK // 16, b_body, 0)

            cps = [
                pltpu.async_copy(
                    feat_hbm.at[gidx_v.at[pl.ds(s * GATHER_SUB, GATHER_SUB)]],
                    rows_v.at[pl.ds(s * GATHER_SUB, GATHER_SUB)], sem)
                for s in range(CHUNK // GATHER_SUB)
            ]
            for cp in cps:
                cp.wait()
            pltpu.sync_copy(rows_v, dense_hbm.at[pl.ds(cb, CHUNK)])
            pltpu.sync_copy(mval_v, mask_hbm.at[pl.ds(cb, CHUNK)])
            return 0
        lax.fori_loop(0, NCHUNK, c_body, 0)

    starts = jnp.repeat(jnp.arange(NW, dtype=jnp.int32) * WSZ, 16)
    return k(starts, indices, features_pad)


def _shift_acc(spad_ref, taps=9):
    acc = None
    for t in range(taps):
        kh, kw = t // 3, t % 3
        delta = (kh - 1) * PW + (kw - 1)
        sl = spad_ref[pl.ds(t * 32, 32), pl.ds(96 + delta, PS)]
        acc = sl if acc is None else acc + sl
    return acc


def _conv3_kernel(x0, x1, x2, w0, w1, w2, spad_ref):
    s = (jnp.dot(w0, x0, preferred_element_type=jnp.float32)
         + jnp.dot(w1, x1, preferred_element_type=jnp.float32)
         + jnp.dot(w2, x2, preferred_element_type=jnp.float32))
    spad_ref[:, pl.ds(96, PS)] = s
    return _shift_acc(spad_ref)


def _zero_halo(spad_ref):
    spad_ref[:, pl.ds(0, 96)] = jnp.zeros((288, 96), jnp.float32)
    spad_ref[:, pl.ds(96 + PS, 96)] = jnp.zeros((288, 96), jnp.float32)


def _stage_a(xpad, mpad, w1t, gscale, beta):
    def body(x0_ref, x1_ref, x2_ref, m_ref, w_ref, g_ref, b_ref, o_ref,
             spad_ref):
        @pl.when(pl.program_id(0) == 0)
        def _():
            _zero_halo(spad_ref)

        acc = _conv3_kernel(x0_ref[0], x1_ref[0], x2_ref[0],
                            w_ref[0], w_ref[1], w_ref[2], spad_ref)
        y = jnp.maximum(acc * g_ref[...] + b_ref[...], 0.0) * m_ref[0]
        o_ref[0] = y

    xspec = lambda j: pl.BlockSpec((1, C_IN, PS), lambda d, j=j: (d + j, 0, 0))
    return pl.pallas_call(
        body,
        grid=(D,),
        in_specs=[
            xspec(0), xspec(1), xspec(2),
            pl.BlockSpec((1, 1, PS), lambda d: (d + 1, 0, 0)),
            pl.BlockSpec((3, 288, C_IN), lambda d: (0, 0, 0)),
            pl.BlockSpec((32, 1), lambda d: (0, 0)),
            pl.BlockSpec((32, 1), lambda d: (0, 0)),
        ],
        out_specs=pl.BlockSpec((1, 32, PS), lambda d: (d, 0, 0)),
        out_shape=jax.ShapeDtypeStruct((D, 32, PS), jnp.float32),
        scratch_shapes=[pltpu.VMEM((288, PS + 192), jnp.float32)],
    )(xpad, xpad, xpad, mpad, w1t, gscale, beta)


def _stage_b(y1p, mpad, w4t):
    def body(x0_ref, x1_ref, x2_ref, m0_ref, m1_ref, m2_ref, w_ref,
             o_ref, mo_ref):
        s = pl.program_id(0)
        interior = jnp.logical_and(s > 0, s < 6)

        @pl.when(interior)
        def _():
            acc = (jnp.dot(w_ref[0], x0_ref[0],
                           preferred_element_type=jnp.float32)
                   + jnp.dot(w_ref[1], x1_ref[0],
                             preferred_element_type=jnp.float32)
                   + jnp.dot(w_ref[2], x2_ref[0],
                             preferred_element_type=jnp.float32))
            o_ref[0] = acc
            mo_ref[0] = jnp.maximum(jnp.maximum(m0_ref[0], m1_ref[0]),
                                    m2_ref[0])

        @pl.when(jnp.logical_not(interior))
        def _():
            o_ref[0] = jnp.zeros((32, PS), jnp.float32)
            mo_ref[0] = jnp.zeros((1, PS), jnp.float32)

    yspec = lambda k: pl.BlockSpec(
        (1, 32, PS),
        lambda s, k=k: (jnp.clip(2 * s - 2 + k, 0, D - 1), 0, 0))
    mspec = lambda k: pl.BlockSpec(
        (1, 1, PS),
        lambda s, k=k: (jnp.clip(2 * s - 1 + k, 0, PD - 1), 0, 0))
    return pl.pallas_call(
        body,
        grid=(7,),
        in_specs=[
            yspec(0), yspec(1), yspec(2),
            mspec(0), mspec(1), mspec(2),
            pl.BlockSpec((3, 32, 32), lambda s: (0, 0, 0)),
        ],
        out_specs=[
            pl.BlockSpec((1, 32, PS), lambda s: (s, 0, 0)),
            pl.BlockSpec((1, 1, PS), lambda s: (s, 0, 0)),
        ],
        out_shape=[
            jax.ShapeDtypeStruct((7, 32, PS), jnp.float32),
            jax.ShapeDtypeStruct((7, 1, PS), jnp.float32),
        ],
    )(y1p, y1p, y1p, mpad, mpad, mpad, w4t)


def _stage_c(ypadded, m2p, wt, nout, dpad_out):
    def body(x0_ref, x1_ref, x2_ref, m_ref, w_ref, o_ref, spad_ref):
        s = pl.program_id(0)

        @pl.when(s == 0)
        def _():
            _zero_halo(spad_ref)

        if dpad_out:
            interior = jnp.logical_and(s > 0, s < nout - 1)

            @pl.when(interior)
            def _():
                acc = _conv3_kernel(x0_ref[0], x1_ref[0], x2_ref[0],
                                    w_ref[0], w_ref[1], w_ref[2], spad_ref)
                o_ref[0] = acc * m_ref[0]

            @pl.when(jnp.logical_not(interior))
            def _():
                o_ref[0] = jnp.zeros((32, PS), jnp.float32)
        else:
            acc = _conv3_kernel(x0_ref[0], x1_ref[0], x2_ref[0],
                                w_ref[0], w_ref[1], w_ref[2], spad_ref)
            o_ref[0] = acc * m_ref[0]

    if dpad_out:
        xidx = lambda k: (lambda s, k=k: (jnp.clip(s - 1 + k, 0, 6), 0, 0))
        midx = lambda s: (jnp.clip(s, 0, 6), 0, 0)
    else:
        xidx = lambda k: (lambda s, k=k: (s + k, 0, 0))
        midx = lambda s: (s + 1, 0, 0)

    return pl.pallas_call(
        body,
        grid=(nout,),
        in_specs=[
            pl.BlockSpec((1, 32, PS), xidx(0)),
            pl.BlockSpec((1, 32, PS), xidx(1)),
            pl.BlockSpec((1, 32, PS), xidx(2)),
            pl.BlockSpec((1, 1, PS), midx),
            pl.BlockSpec((3, 288, 32), lambda s: (0, 0, 0)),
        ],
        out_specs=pl.BlockSpec((1, 32, PS), lambda s: (s, 0, 0)),
        out_shape=jax.ShapeDtypeStruct((nout, 32, PS), jnp.float32),
        scratch_shapes=[pltpu.VMEM((288, PS + 192), jnp.float32)],
    )(ypadded, ypadded, ypadded, m2p, wt)


def _stage_e(y6, w7t):
    def body(x0_ref, x1_ref, x2_ref, w_ref, o_ref):
        acc = (jnp.dot(w_ref[0], x0_ref[0], preferred_element_type=jnp.float32)
               + jnp.dot(w_ref[1], x1_ref[0],
                         preferred_element_type=jnp.float32)
               + jnp.dot(w_ref[2], x2_ref[0],
                         preferred_element_type=jnp.float32))
        o_ref[0] = acc

    yspec = lambda k: pl.BlockSpec((1, 32, PS),
                                   lambda t, k=k: (2 * t + k, 0, 0))
    return pl.pallas_call(
        body,
        grid=(2,),
        in_specs=[yspec(0), yspec(1), yspec(2),
                  pl.BlockSpec((3, 32, 32), lambda t: (0, 0, 0))],
        out_specs=pl.BlockSpec((1, 32, PS), lambda t: (t, 0, 0)),
        out_shape=jax.ShapeDtypeStruct((2, 32, PS), jnp.float32),
    )(y6, y6, y6, w7t)


def kernel(features, indices, w1, gamma, beta, w4, w5, w6, w7):
    features_pad = jnp.concatenate(
        [features, jnp.zeros((N_PADROWS, C_IN), jnp.float32)], axis=0)
    dense_flat, mask_flat = _sc_scatter(indices, features_pad)
    xpad = jnp.transpose(dense_flat[:PVOX].reshape(PD, PS, C_IN), (0, 2, 1))
    mpad = mask_flat[:PVOX].reshape(PD, 1, PS)

    w1t = jnp.transpose(w1, (0, 1, 2, 4, 3)).reshape(3, 288, C_IN)
    w5t = jnp.transpose(w5, (0, 1, 2, 4, 3)).reshape(3, 288, 32)
    w6t = jnp.transpose(w6, (0, 1, 2, 4, 3)).reshape(3, 288, 32)
    w4t = jnp.transpose(w4.reshape(3, 32, 32), (0, 2, 1))
    w7t = jnp.transpose(w7.reshape(3, 32, 32), (0, 2, 1))
    gscale = (gamma / jnp.sqrt(jnp.float32(1.0 + 1e-3))).reshape(32, 1)
    beta2 = beta.reshape(32, 1)

    y1p = _stage_a(xpad, mpad, w1t, gscale, beta2)
    y4p, m2p = _stage_b(y1p, mpad, w4t)
    y5p = _stage_c(y4p, m2p, w5t, 7, True)
    y6 = _stage_c(y5p, m2p, w6t, 5, False)
    y7 = _stage_e(y6, w7t)

    out = y7.reshape(2, 32, PH, PW)[:, :, 1:H + 1, 1:W + 1]
    return jnp.transpose(out, (1, 0, 2, 3))[None]
